# Initial kernel scaffold; baseline (speedup 1.0000x reference)
import jax
import jax.numpy as jnp
from jax.experimental import pallas as pl


def kernel(x, e, g, edges, edge_idx, node_idx, W_edge, b_edge, W_node, b_node):
    raise NotImplementedError("write your pallas kernel here")


# speedup vs baseline: 11.5961x; 11.5961x over previous
import functools

import jax
import jax.numpy as jnp
from jax import lax
from jax.experimental import pallas as pl
from jax.experimental.pallas import tpu as pltpu
from jax.experimental.pallas import tpu_sc as plsc

N = 10000
E = 320000
D = 128
DE = 16
G = 16

NC = 2
NS = 16
NW = NC * NS
EPW = E // NW
LANES = 16
VECS = EPW // LANES
NVECS = N // LANES

ER = E // D
EK = D * DE
KB = 256


def _proj_body(e2_ref, webig_ref, x_ref, wx_ref, g_ref, wg_ref, be_ref,
               pe2_ref, xp_ref, pgp_ref):
    i = pl.program_id(0)
    part = jnp.dot(e2_ref[...], webig_ref[...],
                   preferred_element_type=jnp.float32)

    @pl.when(i == 0)
    def _():
        pe2_ref[...] = part + be_ref[0, 0]

    @pl.when(i != 0)
    def _():
        pe2_ref[...] += part

    @pl.when(i == 0)
    def _():
        xp_ref[...] = jnp.dot(x_ref[...], wx_ref[...],
                              preferred_element_type=jnp.float32)
        pgp_ref[...] = jnp.dot(g_ref[...], wg_ref[...],
                               preferred_element_type=jnp.float32)


def _projections(e2, we_big, x, wx, g, wg, be):
    return pl.pallas_call(
        _proj_body,
        grid=(EK // KB,),
        in_specs=[
            pl.BlockSpec((ER, KB), lambda i: (0, i)),
            pl.BlockSpec((KB, D), lambda i: (i, 0)),
            pl.BlockSpec((N, D), lambda i: (0, 0)),
            pl.BlockSpec((D, 3), lambda i: (0, 0)),
            pl.BlockSpec((G, 64), lambda i: (0, 0)),
            pl.BlockSpec((64, 1), lambda i: (0, 0)),
            pl.BlockSpec(memory_space=pltpu.SMEM),
        ],
        out_specs=[
            pl.BlockSpec((ER, D), lambda i: (0, 0)),
            pl.BlockSpec((N, 3), lambda i: (0, 0)),
            pl.BlockSpec((G, 1), lambda i: (0, 0)),
        ],
        out_shape=[
            jax.ShapeDtypeStruct((ER, D), jnp.float32),
            jax.ShapeDtypeStruct((N, 3), jnp.float32),
            jax.ShapeDtypeStruct((G, 1), jnp.float32),
        ],
    )(e2, we_big, x, wx, g, wg, be)


def _edge_body(src_hbm, dst_hbm, ei_hbm, pe_hbm, p2_hbm, p3_hbm, pg_hbm,
               xw_hbm, out_hbm,
               src_v, dst_v, ei_v, pe_v, p2_v, p3_v, pg_v, acc_v):
    wid = lax.axis_index("s") * NC + lax.axis_index("c")
    base = wid * EPW

    pltpu.sync_copy(src_hbm.at[pl.ds(base, EPW)], src_v)
    pltpu.sync_copy(dst_hbm.at[pl.ds(base, EPW)], dst_v)
    pltpu.sync_copy(ei_hbm.at[pl.ds(base, EPW)], ei_v)
    pltpu.sync_copy(pe_hbm.at[pl.ds(base, EPW)], pe_v)
    pltpu.sync_copy(p2_hbm, p2_v)
    pltpu.sync_copy(p3_hbm, p3_v)
    pltpu.sync_copy(pg_hbm, pg_v)

    @pl.when(wid == 0)
    def _():
        pltpu.sync_copy(xw_hbm, acc_v)

    @pl.when(wid != 0)
    def _():
        zeros = jnp.zeros((LANES,), jnp.float32)

        def zbody(j, _):
            acc_v[pl.ds(j * LANES, LANES)] = zeros
            return 0

        lax.fori_loop(0, NVECS, zbody, 0)

    def body(j, _):
        sl = pl.ds(j * LANES, LANES)
        s = src_v[sl]
        d = dst_v[sl]
        gi = ei_v[sl]
        v = pe_v[sl]
        v = v + plsc.load_gather(p2_v, [s])
        v = v + plsc.load_gather(p3_v, [d])
        v = v + plsc.load_gather(pg_v, [gi])
        plsc.addupdate_scatter(acc_v, [d], v)
        return 0

    lax.fori_loop(0, VECS, body, 0)

    pltpu.sync_copy(acc_v, out_hbm.at[wid])


@functools.partial(
    pl.kernel,
    out_type=jax.ShapeDtypeStruct((NW, N), jnp.float32),
    mesh=plsc.VectorSubcoreMesh(core_axis_name="c", subcore_axis_name="s",
                                num_cores=NC, num_subcores=NS),
    compiler_params=pltpu.CompilerParams(needs_layout_passes=False),
    scratch_types=[
        pltpu.VMEM((EPW,), jnp.int32),
        pltpu.VMEM((EPW,), jnp.int32),
        pltpu.VMEM((EPW,), jnp.int32),
        pltpu.VMEM((EPW,), jnp.float32),
        pltpu.VMEM((N,), jnp.float32),
        pltpu.VMEM((N,), jnp.float32),
        pltpu.VMEM((G,), jnp.float32),
        pltpu.VMEM((N,), jnp.float32),
    ],
)
def _edge_agg(src, dst, ei, pe, p2, p3, pg, xw, out,
              src_v, dst_v, ei_v, pe_v, p2_v, p3_v, pg_v, acc_v):
    _edge_body(src, dst, ei, pe, p2, p3, pg, xw, out,
               src_v, dst_v, ei_v, pe_v, p2_v, p3_v, pg_v, acc_v)


def _reduce_body(p_ref, bn_ref, out_ref):
    out_ref[...] = (
        jnp.sum(p_ref[...], axis=0, keepdims=True) + bn_ref[0, 0]
    )


def _reduce_partials(partials, bn):
    return pl.pallas_call(
        _reduce_body,
        in_specs=[
            pl.BlockSpec((NW, N), lambda: (0, 0)),
            pl.BlockSpec(memory_space=pltpu.SMEM),
        ],
        out_specs=pl.BlockSpec((1, N), lambda: (0, 0)),
        out_shape=jax.ShapeDtypeStruct((1, N), jnp.float32),
    )(partials, bn)


def kernel(x, e, g, edges, edge_idx, node_idx, W_edge, b_edge, W_node, b_node):
    del node_idx
    w_last = W_node[D, 0]

    we = W_edge[:DE, :] * w_last
    we_big = (jnp.eye(D, dtype=jnp.float32)[:, None, :]
              * we[None, :, 0:1]).reshape(EK, D)
    wx = jnp.concatenate(
        [W_edge[DE:DE + D, :] * w_last,
         W_edge[DE + D:DE + 2 * D, :] * w_last,
         W_node[:D, :]], axis=1)
    wg = W_edge[DE + 2 * D:, :] * w_last
    be = (b_edge * w_last).reshape(1, 1)
    bn = b_node.reshape(1, 1)

    e2 = e.reshape(ER, EK)
    pe2, xp, pgp = _projections(e2, we_big, x, wx, g, wg, be)

    pe = pe2.reshape(E)
    p2 = xp[:, 0]
    p3 = xp[:, 1]
    xw = xp[:, 2]
    pg = pgp.reshape(G)

    partials = _edge_agg(edges[0], edges[1], edge_idx, pe, p2, p3, pg, xw)
    out = _reduce_partials(partials, bn)
    return out.reshape(N, 1)


# speedup vs baseline: 11.8349x; 1.0206x over previous
import functools

import jax
import jax.numpy as jnp
from jax import lax
from jax.experimental import pallas as pl
from jax.experimental.pallas import tpu as pltpu
from jax.experimental.pallas import tpu_sc as plsc

N = 10000
E = 320000
D = 128
DE = 16
G = 16

NC = 2
NS = 16
NW = NC * NS
EPW = E // NW
LANES = 16
VECS = EPW // LANES
NVECS = N // LANES

EB = EPW


def _proj_body(e_ref, x_ref, g_ref, wev_ref, wnv_ref, wns_ref, be_ref,
               pe_ref, p2_ref, p3_ref, xw_ref, pg_ref):
    i = pl.program_id(0)
    wl = wns_ref[D, 0]

    we_col = wev_ref[0:DE, :]
    y = lax.dot_general(we_col, e_ref[...],
                        (((0,), (1,)), ((), ())),
                        preferred_element_type=jnp.float32)
    pe_ref[pl.ds(i, 1), :] = y * wl + be_ref[0, 0] * wl

    @pl.when(i == 0)
    def _():
        xv = x_ref[...]
        wa = wev_ref[DE:DE + D, :]
        wb = wev_ref[DE + D:DE + 2 * D, :]
        wn = wnv_ref[0:D, :]
        wg = wev_ref[DE + 2 * D:DE + 2 * D + 64, :]
        p2_ref[...] = jnp.dot(xv, wa, preferred_element_type=jnp.float32) * wl
        p3_ref[...] = jnp.dot(xv, wb, preferred_element_type=jnp.float32) * wl
        xw_ref[...] = jnp.dot(xv, wn, preferred_element_type=jnp.float32)
        pg_ref[...] = jnp.dot(g_ref[...], wg,
                              preferred_element_type=jnp.float32) * wl


def _projections(e, x, g, W_edge, W_node, be):
    return pl.pallas_call(
        _proj_body,
        grid=(E // EB,),
        in_specs=[
            pl.BlockSpec((EB, DE), lambda i: (i, 0)),
            pl.BlockSpec((N, D), lambda i: (0, 0)),
            pl.BlockSpec((G, 64), lambda i: (0, 0)),
            pl.BlockSpec((DE + 2 * D + 64, 1), lambda i: (0, 0)),
            pl.BlockSpec((D + 1, 1), lambda i: (0, 0)),
            pl.BlockSpec(memory_space=pltpu.SMEM),
            pl.BlockSpec(memory_space=pltpu.SMEM),
        ],
        out_specs=[
            pl.BlockSpec((NW, EPW), lambda i: (0, 0)),
            pl.BlockSpec((N, 1), lambda i: (0, 0)),
            pl.BlockSpec((N, 1), lambda i: (0, 0)),
            pl.BlockSpec((N, 1), lambda i: (0, 0)),
            pl.BlockSpec((G, 1), lambda i: (0, 0)),
        ],
        out_shape=[
            jax.ShapeDtypeStruct((NW, EPW), jnp.float32),
            jax.ShapeDtypeStruct((N, 1), jnp.float32),
            jax.ShapeDtypeStruct((N, 1), jnp.float32),
            jax.ShapeDtypeStruct((N, 1), jnp.float32),
            jax.ShapeDtypeStruct((G, 1), jnp.float32),
        ],
    )(e, x, g, W_edge, W_node, W_node, be)


def _edge_body(src_hbm, dst_hbm, ei_hbm, pe_hbm, p2_hbm, p3_hbm, pg_hbm,
               xw_hbm, out_hbm,
               src_v, dst_v, ei_v, pe_v, p2_v, p3_v, pg_v, acc_v):
    wid = lax.axis_index("s") * NC + lax.axis_index("c")
    base = wid * EPW

    pltpu.sync_copy(src_hbm.at[pl.ds(base, EPW)], src_v)
    pltpu.sync_copy(dst_hbm.at[pl.ds(base, EPW)], dst_v)
    pltpu.sync_copy(ei_hbm.at[pl.ds(base, EPW)], ei_v)
    pltpu.sync_copy(pe_hbm.at[wid], pe_v)
    pltpu.sync_copy(p2_hbm, p2_v)
    pltpu.sync_copy(p3_hbm, p3_v)
    pltpu.sync_copy(pg_hbm, pg_v)

    @pl.when(wid == 0)
    def _():
        pltpu.sync_copy(xw_hbm, acc_v)

    @pl.when(wid != 0)
    def _():
        zeros = jnp.zeros((LANES,), jnp.float32)

        def zbody(j, _):
            acc_v[pl.ds(j * LANES, LANES)] = zeros
            return 0

        lax.fori_loop(0, NVECS, zbody, 0)

    def body(j, _):
        sl = pl.ds(j * LANES, LANES)
        s = src_v[sl]
        d = dst_v[sl]
        gi = ei_v[sl]
        v = pe_v[sl]
        v = v + plsc.load_gather(p2_v, [s])
        v = v + plsc.load_gather(p3_v, [d])
        v = v + plsc.load_gather(pg_v, [gi])
        plsc.addupdate_scatter(acc_v, [d], v)
        return 0

    lax.fori_loop(0, VECS, body, 0)

    pltpu.sync_copy(acc_v, out_hbm.at[wid])


@functools.partial(
    pl.kernel,
    out_type=jax.ShapeDtypeStruct((NW, N), jnp.float32),
    mesh=plsc.VectorSubcoreMesh(core_axis_name="c", subcore_axis_name="s",
                                num_cores=NC, num_subcores=NS),
    compiler_params=pltpu.CompilerParams(needs_layout_passes=False),
    scratch_types=[
        pltpu.VMEM((EPW,), jnp.int32),
        pltpu.VMEM((EPW,), jnp.int32),
        pltpu.VMEM((EPW,), jnp.int32),
        pltpu.VMEM((EPW,), jnp.float32),
        pltpu.VMEM((N,), jnp.float32),
        pltpu.VMEM((N,), jnp.float32),
        pltpu.VMEM((G,), jnp.float32),
        pltpu.VMEM((N,), jnp.float32),
    ],
)
def _edge_agg(src, dst, ei, pe, p2, p3, pg, xw, out,
              src_v, dst_v, ei_v, pe_v, p2_v, p3_v, pg_v, acc_v):
    _edge_body(src, dst, ei, pe, p2, p3, pg, xw, out,
               src_v, dst_v, ei_v, pe_v, p2_v, p3_v, pg_v, acc_v)


def _reduce_body(p_ref, bn_ref, out_ref):
    out_ref[...] = (
        jnp.sum(p_ref[...], axis=0, keepdims=True) + bn_ref[0, 0]
    )


def _reduce_partials(partials, bn):
    return pl.pallas_call(
        _reduce_body,
        in_specs=[
            pl.BlockSpec((NW, N), lambda: (0, 0)),
            pl.BlockSpec(memory_space=pltpu.SMEM),
        ],
        out_specs=pl.BlockSpec((1, N), lambda: (0, 0)),
        out_shape=jax.ShapeDtypeStruct((1, N), jnp.float32),
    )(partials, bn)


def kernel(x, e, g, edges, edge_idx, node_idx, W_edge, b_edge, W_node, b_node):
    del node_idx
    be = b_edge.reshape(1, 1)
    bn = b_node.reshape(1, 1)

    pe, p2, p3, xw, pgp = _projections(e, x, g, W_edge, W_node, be)

    partials = _edge_agg(edges[0], edges[1], edge_idx,
                         pe, p2.reshape(N), p3.reshape(N), pgp.reshape(G),
                         xw.reshape(N))
    out = _reduce_partials(partials, bn)
    return out.reshape(N, 1)


# speedup vs baseline: 22.2273x; 1.8781x over previous
import functools

import jax
import jax.numpy as jnp
from jax import lax
from jax.experimental import pallas as pl
from jax.experimental.pallas import tpu as pltpu
from jax.experimental.pallas import tpu_sc as plsc

N = 10000
E = 320000
D = 128
DE = 16
G = 16

NC = 2
NS = 16
NW = NC * NS
EPW = E // NW
LANES = 16
NVECS = N // LANES
VECS = EPW // LANES

ROWS = 5
EB = E // ROWS


def _proj_body(et_ref, edg_ref, ei_ref, x_ref, g_ref, wev_ref, wnv_ref,
               wns_ref, be_ref,
               pe_ref, src_ref, dst_ref, eio_ref, xp_ref, pg_ref):
    i = pl.program_id(0)
    wl = wns_ref[D, 0]
    wes = wev_ref[...] * wl

    we_col = wes[0:DE, :]
    y = lax.dot_general(we_col, et_ref[...],
                        (((0,), (0,)), ((), ())),
                        preferred_element_type=jnp.float32)
    pe_ref[...] = y + be_ref[0, 0] * wl

    src_ref[...] = edg_ref[0:1, :]
    dst_ref[...] = edg_ref[1:2, :]
    eio_ref[...] = ei_ref[...]

    @pl.when(i == 0)
    def _():
        wx = jnp.concatenate(
            [wes[DE:DE + D, :], wes[DE + D:DE + 2 * D, :],
             wnv_ref[0:D, :]], axis=1)
        xp_ref[...] = lax.dot_general(wx, x_ref[...],
                                      (((0,), (1,)), ((), ())),
                                      preferred_element_type=jnp.float32)
        wg = wes[DE + 2 * D:DE + 2 * D + 64, :]
        pg_ref[...] = lax.dot_general(wg, g_ref[...],
                                      (((0,), (1,)), ((), ())),
                                      preferred_element_type=jnp.float32)


def _projections(et, edges, ei2, x, g, W_edge, W_node, be):
    return pl.pallas_call(
        _proj_body,
        grid=(ROWS,),
        in_specs=[
            pl.BlockSpec((DE, EB), lambda i: (0, i)),
            pl.BlockSpec((2, EB), lambda i: (0, i)),
            pl.BlockSpec((1, EB), lambda i: (0, i)),
            pl.BlockSpec((N, D), lambda i: (0, 0)),
            pl.BlockSpec((G, 64), lambda i: (0, 0)),
            pl.BlockSpec((DE + 2 * D + 64, 1), lambda i: (0, 0)),
            pl.BlockSpec((D + 1, 1), lambda i: (0, 0)),
            pl.BlockSpec(memory_space=pltpu.SMEM),
            pl.BlockSpec(memory_space=pltpu.SMEM),
        ],
        out_specs=[
            pl.BlockSpec((1, EB), lambda i: (0, i)),
            pl.BlockSpec((1, EB), lambda i: (0, i)),
            pl.BlockSpec((1, EB), lambda i: (0, i)),
            pl.BlockSpec((1, EB), lambda i: (0, i)),
            pl.BlockSpec((3, N), lambda i: (0, 0)),
            pl.BlockSpec((1, G), lambda i: (0, 0)),
        ],
        out_shape=[
            jax.ShapeDtypeStruct((1, E), jnp.float32),
            jax.ShapeDtypeStruct((1, E), jnp.int32),
            jax.ShapeDtypeStruct((1, E), jnp.int32),
            jax.ShapeDtypeStruct((1, E), jnp.int32),
            jax.ShapeDtypeStruct((3, N), jnp.float32),
            jax.ShapeDtypeStruct((1, G), jnp.float32),
        ],
    )(et, edges, ei2, x, g, W_edge, W_node, W_node, be)


def _edge_body(src_hbm, dst_hbm, ei_hbm, pe_hbm, p2_hbm, p3_hbm, pg_hbm,
               xw_hbm, out_hbm,
               src_v, dst_v, ei_v, pe_v, p2_v, p3_v, pg_v, acc_v):
    wid = lax.axis_index("s") * NC + lax.axis_index("c")
    base = wid * EPW

    pltpu.sync_copy(src_hbm.at[pl.ds(base, EPW)], src_v)
    pltpu.sync_copy(dst_hbm.at[pl.ds(base, EPW)], dst_v)
    pltpu.sync_copy(ei_hbm.at[pl.ds(base, EPW)], ei_v)
    pltpu.sync_copy(pe_hbm.at[pl.ds(base, EPW)], pe_v)
    pltpu.sync_copy(p2_hbm, p2_v)
    pltpu.sync_copy(p3_hbm, p3_v)
    pltpu.sync_copy(pg_hbm, pg_v)

    @pl.when(wid == 0)
    def _():
        pltpu.sync_copy(xw_hbm, acc_v)

    @pl.when(wid != 0)
    def _():
        zeros = jnp.zeros((LANES,), jnp.float32)

        def zbody(j, _):
            acc_v[pl.ds(j * LANES, LANES)] = zeros
            return 0

        lax.fori_loop(0, NVECS, zbody, 0)

    def body(j, _):
        sl = pl.ds(j * LANES, LANES)
        s = src_v[sl]
        d = dst_v[sl]
        gi = ei_v[sl]
        v = pe_v[sl]
        v = v + plsc.load_gather(p2_v, [s])
        v = v + plsc.load_gather(p3_v, [d])
        v = v + plsc.load_gather(pg_v, [gi])
        plsc.addupdate_scatter(acc_v, [d], v)
        return 0

    lax.fori_loop(0, VECS, body, 0)

    pltpu.sync_copy(acc_v, out_hbm.at[wid])


@functools.partial(
    pl.kernel,
    out_type=jax.ShapeDtypeStruct((NW, N), jnp.float32),
    mesh=plsc.VectorSubcoreMesh(core_axis_name="c", subcore_axis_name="s",
                                num_cores=NC, num_subcores=NS),
    compiler_params=pltpu.CompilerParams(needs_layout_passes=False),
    scratch_types=[
        pltpu.VMEM((EPW,), jnp.int32),
        pltpu.VMEM((EPW,), jnp.int32),
        pltpu.VMEM((EPW,), jnp.int32),
        pltpu.VMEM((EPW,), jnp.float32),
        pltpu.VMEM((N,), jnp.float32),
        pltpu.VMEM((N,), jnp.float32),
        pltpu.VMEM((G,), jnp.float32),
        pltpu.VMEM((N,), jnp.float32),
    ],
)
def _edge_agg(src, dst, ei, pe, p2, p3, pg, xw, out,
              src_v, dst_v, ei_v, pe_v, p2_v, p3_v, pg_v, acc_v):
    _edge_body(src, dst, ei, pe, p2, p3, pg, xw, out,
               src_v, dst_v, ei_v, pe_v, p2_v, p3_v, pg_v, acc_v)


def _reduce_body(p_ref, bn_ref, out_ref):
    out_ref[...] = (
        jnp.sum(p_ref[...], axis=0, keepdims=True) + bn_ref[0, 0]
    )


def _reduce_partials(partials, bn):
    return pl.pallas_call(
        _reduce_body,
        in_specs=[
            pl.BlockSpec((NW, N), lambda: (0, 0)),
            pl.BlockSpec(memory_space=pltpu.SMEM),
        ],
        out_specs=pl.BlockSpec((1, N), lambda: (0, 0)),
        out_shape=jax.ShapeDtypeStruct((1, N), jnp.float32),
    )(partials, bn)


def kernel(x, e, g, edges, edge_idx, node_idx, W_edge, b_edge, W_node, b_node):
    del node_idx
    be = b_edge.reshape(1, 1)
    bn = b_node.reshape(1, 1)

    pe, src2, dst2, eio, xp, pgp = _projections(
        e.T, edges, edge_idx.reshape(1, E), x, g, W_edge, W_node, be)

    partials = _edge_agg(src2.reshape(E), dst2.reshape(E), eio.reshape(E),
                         pe.reshape(E), xp[0], xp[1], pgp.reshape(G), xp[2])
    out = _reduce_partials(partials, bn)
    return out.reshape(N, 1)


# speedup vs baseline: 44.8047x; 2.0158x over previous
import functools

import jax
import jax.numpy as jnp
from jax import lax
from jax.experimental import pallas as pl
from jax.experimental.pallas import tpu as pltpu
from jax.experimental.pallas import tpu_sc as plsc

N = 10000
E = 320000
D = 128
DE = 16
G = 16

NC = 2
NS = 16
NW = NC * NS
EPW = E // NW
LANES = 16
NVECS = N // LANES
VECS = EPW // LANES

ROWS = 5
EB = E // ROWS
EPWP = EPW + 112


def _proj_body(et_ref, edg_ref, x_ref, g_ref, wev_ref, wnv_ref,
               wns_ref, be_ref,
               pe_ref, src_ref, dst_ref, xp_ref, pg_ref):
    i = pl.program_id(0)
    wl = wns_ref[D, 0]
    wes = wev_ref[...] * wl

    we_col = wes[0:DE, :]
    y = lax.dot_general(we_col, et_ref[...],
                        (((0,), (0,)), ((), ())),
                        preferred_element_type=jnp.float32)
    pe_ref[...] = y + be_ref[0, 0] * wl

    src_ref[...] = edg_ref[0:1, :]
    dst_ref[...] = edg_ref[1:2, :]

    @pl.when(i == 0)
    def _():
        wx = jnp.concatenate(
            [wes[DE:DE + D, :], wes[DE + D:DE + 2 * D, :],
             wnv_ref[0:D, :]], axis=1)
        xp_ref[...] = lax.dot_general(wx, x_ref[...],
                                      (((0,), (1,)), ((), ())),
                                      preferred_element_type=jnp.float32)
        wg = wes[DE + 2 * D:DE + 2 * D + 64, :]
        pg_ref[...] = lax.dot_general(wg, g_ref[...],
                                      (((0,), (1,)), ((), ())),
                                      preferred_element_type=jnp.float32)


def _projections(et, edges, x, g, W_edge, W_node, be):
    return pl.pallas_call(
        _proj_body,
        grid=(ROWS,),
        in_specs=[
            pl.BlockSpec((DE, EB), lambda i: (0, i)),
            pl.BlockSpec((2, EB), lambda i: (0, i)),
            pl.BlockSpec((N, D), lambda i: (0, 0)),
            pl.BlockSpec((G, 64), lambda i: (0, 0)),
            pl.BlockSpec((DE + 2 * D + 64, 1), lambda i: (0, 0)),
            pl.BlockSpec((D + 1, 1), lambda i: (0, 0)),
            pl.BlockSpec(memory_space=pltpu.SMEM),
            pl.BlockSpec(memory_space=pltpu.SMEM),
        ],
        out_specs=[
            pl.BlockSpec((1, EB), lambda i: (0, i)),
            pl.BlockSpec((1, EB), lambda i: (0, i)),
            pl.BlockSpec((1, EB), lambda i: (0, i)),
            pl.BlockSpec((3, N), lambda i: (0, 0)),
            pl.BlockSpec((1, G), lambda i: (0, 0)),
        ],
        out_shape=[
            jax.ShapeDtypeStruct((1, E), jnp.float32),
            jax.ShapeDtypeStruct((1, E), jnp.int32),
            jax.ShapeDtypeStruct((1, E), jnp.int32),
            jax.ShapeDtypeStruct((3, N), jnp.float32),
            jax.ShapeDtypeStruct((1, G), jnp.float32),
        ],
    )(et, edges, x, g, W_edge, W_node, W_node, be)


def _edge_body(src_hbm, dst_hbm, ei_hbm, pe_hbm, p2_hbm, p3_hbm, pg_hbm,
               xw_hbm, out_hbm,
               src_v, dst_v, ei_v, pe_v, p2_v, p3_v, pg_v, acc_v):
    wid = lax.axis_index("s") * NC + lax.axis_index("c")
    base = wid * EPW
    col0 = pl.multiple_of((base // 128) * 128, 128)
    delta = base - col0

    pltpu.sync_copy(src_hbm.at[0, pl.ds(col0, EPWP)], src_v)
    pltpu.sync_copy(dst_hbm.at[0, pl.ds(col0, EPWP)], dst_v)
    pltpu.sync_copy(ei_hbm.at[pl.ds(base, EPW)], ei_v)
    pltpu.sync_copy(pe_hbm.at[0, pl.ds(col0, EPWP)], pe_v)
    pltpu.sync_copy(p2_hbm, p2_v)
    pltpu.sync_copy(p3_hbm, p3_v)
    pltpu.sync_copy(pg_hbm, pg_v)

    @pl.when(wid == 0)
    def _():
        pltpu.sync_copy(xw_hbm, acc_v)

    @pl.when(wid != 0)
    def _():
        zeros = jnp.zeros((LANES,), jnp.float32)

        def zbody(j, _):
            acc_v[pl.ds(j * LANES, LANES)] = zeros
            return 0

        lax.fori_loop(0, NVECS, zbody, 0)

    def body(j, _):
        sl = pl.ds(delta + j * LANES, LANES)
        s = src_v[sl]
        d = dst_v[sl]
        gi = ei_v[pl.ds(j * LANES, LANES)]
        v = pe_v[sl]
        v = v + plsc.load_gather(p2_v, [s])
        v = v + plsc.load_gather(p3_v, [d])
        v = v + plsc.load_gather(pg_v, [gi])
        plsc.addupdate_scatter(acc_v, [d], v)
        return 0

    lax.fori_loop(0, VECS, body, 0)

    pltpu.sync_copy(acc_v, out_hbm.at[wid])


@functools.partial(
    pl.kernel,
    out_type=jax.ShapeDtypeStruct((NW, N), jnp.float32),
    mesh=plsc.VectorSubcoreMesh(core_axis_name="c", subcore_axis_name="s",
                                num_cores=NC, num_subcores=NS),
    compiler_params=pltpu.CompilerParams(needs_layout_passes=False),
    scratch_types=[
        pltpu.VMEM((EPWP,), jnp.int32),
        pltpu.VMEM((EPWP,), jnp.int32),
        pltpu.VMEM((EPW,), jnp.int32),
        pltpu.VMEM((EPWP,), jnp.float32),
        pltpu.VMEM((N,), jnp.float32),
        pltpu.VMEM((N,), jnp.float32),
        pltpu.VMEM((G,), jnp.float32),
        pltpu.VMEM((N,), jnp.float32),
    ],
)
def _edge_agg(src, dst, ei, pe, p2, p3, pg, xw, out,
              src_v, dst_v, ei_v, pe_v, p2_v, p3_v, pg_v, acc_v):
    _edge_body(src, dst, ei, pe, p2, p3, pg, xw, out,
               src_v, dst_v, ei_v, pe_v, p2_v, p3_v, pg_v, acc_v)


def _reduce_body(p_ref, bn_ref, out_ref):
    out_ref[...] = (
        jnp.sum(p_ref[...], axis=0, keepdims=True) + bn_ref[0, 0]
    )


def _reduce_partials(partials, bn):
    return pl.pallas_call(
        _reduce_body,
        in_specs=[
            pl.BlockSpec((NW, N), lambda: (0, 0)),
            pl.BlockSpec(memory_space=pltpu.SMEM),
        ],
        out_specs=pl.BlockSpec((1, N), lambda: (0, 0)),
        out_shape=jax.ShapeDtypeStruct((1, N), jnp.float32),
    )(partials, bn)


def kernel(x, e, g, edges, edge_idx, node_idx, W_edge, b_edge, W_node, b_node):
    del node_idx
    be = b_edge.reshape(1, 1)
    bn = b_node.reshape(1, 1)

    pe, src2, dst2, xp, pgp = _projections(
        e.T, edges, x, g, W_edge, W_node, be)

    partials = _edge_agg(src2, dst2, edge_idx,
                         pe, xp[0], xp[1], pgp.reshape(G), xp[2])
    out = _reduce_partials(partials, bn)
    return out.reshape(N, 1)


# speedup vs baseline: 55.4893x; 1.2385x over previous
import functools

import jax
import jax.numpy as jnp
from jax import lax
from jax.experimental import pallas as pl
from jax.experimental.pallas import tpu as pltpu
from jax.experimental.pallas import tpu_sc as plsc

N = 10000
E = 320000
D = 128
DE = 16
G = 16

NC = 2
NS = 16
NW = NC * NS
EPW = E // NW
LANES = 16
NVECS = N // LANES
VECS = EPW // LANES

ROWS = 5
EB = E // ROWS
EPWP = EPW + 112
NP = N + 112


def _proj_body(et_ref, edg_ref, x_ref, g_ref, wev_ref, wnv_ref,
               wns_ref, be_ref,
               pe_ref, src_ref, dst_ref, p2_ref, p3_ref, xw_ref, pg_ref):
    i = pl.program_id(0)
    wl = wns_ref[D, 0]
    wes = wev_ref[...] * wl

    we_col = wes[0:DE, :]
    y = lax.dot_general(we_col, et_ref[...],
                        (((0,), (0,)), ((), ())),
                        preferred_element_type=jnp.float32)
    pe_ref[...] = y + be_ref[0, 0] * wl

    src_ref[...] = edg_ref[0:1, :]
    dst_ref[...] = edg_ref[1:2, :]

    @pl.when(i == 0)
    def _():
        wx = jnp.concatenate(
            [wes[DE:DE + D, :], wes[DE + D:DE + 2 * D, :],
             wnv_ref[0:D, :]], axis=1)
        xp = lax.dot_general(wx, x_ref[...],
                             (((0,), (1,)), ((), ())),
                             preferred_element_type=jnp.float32)
        p2_ref[0:1, 0:N] = xp[0:1, :]
        p3_ref[0:1, 0:N] = xp[1:2, :]
        xw_ref[0:1, 0:N] = xp[2:3, :]
        wg = wes[DE + 2 * D:DE + 2 * D + 64, :]
        pg_ref[...] = lax.dot_general(wg, g_ref[...],
                                      (((0,), (1,)), ((), ())),
                                      preferred_element_type=jnp.float32)


def _projections(et, edges, x, g, W_edge, W_node, be):
    return pl.pallas_call(
        _proj_body,
        grid=(ROWS,),
        in_specs=[
            pl.BlockSpec((DE, EB), lambda i: (0, i)),
            pl.BlockSpec((2, EB), lambda i: (0, i)),
            pl.BlockSpec((N, D), lambda i: (0, 0)),
            pl.BlockSpec((G, 64), lambda i: (0, 0)),
            pl.BlockSpec((DE + 2 * D + 64, 1), lambda i: (0, 0)),
            pl.BlockSpec((D + 1, 1), lambda i: (0, 0)),
            pl.BlockSpec(memory_space=pltpu.SMEM),
            pl.BlockSpec(memory_space=pltpu.SMEM),
        ],
        out_specs=[
            pl.BlockSpec((1, EB), lambda i: (0, i)),
            pl.BlockSpec((1, EB), lambda i: (0, i)),
            pl.BlockSpec((1, EB), lambda i: (0, i)),
            pl.BlockSpec((1, NP), lambda i: (0, 0)),
            pl.BlockSpec((1, NP), lambda i: (0, 0)),
            pl.BlockSpec((1, NP), lambda i: (0, 0)),
            pl.BlockSpec((1, G), lambda i: (0, 0)),
        ],
        out_shape=[
            jax.ShapeDtypeStruct((1, E), jnp.float32),
            jax.ShapeDtypeStruct((1, E), jnp.int32),
            jax.ShapeDtypeStruct((1, E), jnp.int32),
            jax.ShapeDtypeStruct((1, NP), jnp.float32),
            jax.ShapeDtypeStruct((1, NP), jnp.float32),
            jax.ShapeDtypeStruct((1, NP), jnp.float32),
            jax.ShapeDtypeStruct((1, G), jnp.float32),
        ],
    )(et, edges, x, g, W_edge, W_node, W_node, be)


def _edge_body(src_hbm, dst_hbm, ei_hbm, pe_hbm, p2_hbm, p3_hbm, pg_hbm,
               xw_hbm, out_hbm,
               src_v, dst_v, ei_v, pe_v, p2_v, p3_v, pg_v, acc_v, sem):
    wid = lax.axis_index("s") * NC + lax.axis_index("c")
    base = wid * EPW
    col0 = pl.multiple_of((base // 128) * 128, 128)
    delta = base - col0

    cps = [
        pltpu.async_copy(src_hbm.at[0, pl.ds(col0, EPWP)], src_v, sem),
        pltpu.async_copy(dst_hbm.at[0, pl.ds(col0, EPWP)], dst_v, sem),
        pltpu.async_copy(ei_hbm.at[pl.ds(base, EPW)], ei_v, sem),
        pltpu.async_copy(pe_hbm.at[0, pl.ds(col0, EPWP)], pe_v, sem),
        pltpu.async_copy(p2_hbm.at[0, pl.ds(0, NP)], p2_v, sem),
        pltpu.async_copy(p3_hbm.at[0, pl.ds(0, NP)], p3_v, sem),
        pltpu.async_copy(pg_hbm.at[0, pl.ds(0, G)], pg_v, sem),
    ]

    @pl.when(wid == 0)
    def _():
        pltpu.sync_copy(xw_hbm.at[0, pl.ds(0, NP)], acc_v)

    @pl.when(wid != 0)
    def _():
        zeros = jnp.zeros((LANES,), jnp.float32)

        @plsc.parallel_loop(0, NP // LANES, 1, unroll=8)
        def zbody(j):
            acc_v[pl.ds(j * LANES, LANES)] = zeros

    for c in cps:
        c.wait()

    @plsc.parallel_loop(0, VECS, 1, unroll=4)
    def body(j):
        sl = pl.ds(delta + j * LANES, LANES)
        s = src_v[sl]
        d = dst_v[sl]
        gi = ei_v[pl.ds(j * LANES, LANES)]
        v = pe_v[sl]
        v = v + plsc.load_gather(p2_v, [s])
        v = v + plsc.load_gather(p3_v, [d])
        v = v + plsc.load_gather(pg_v, [gi])
        plsc.addupdate_scatter(acc_v, [d], v)

    pltpu.sync_copy(acc_v, out_hbm.at[wid])


@functools.partial(
    pl.kernel,
    out_type=jax.ShapeDtypeStruct((NW, NP), jnp.float32),
    mesh=plsc.VectorSubcoreMesh(core_axis_name="c", subcore_axis_name="s",
                                num_cores=NC, num_subcores=NS),
    compiler_params=pltpu.CompilerParams(needs_layout_passes=False),
    scratch_types=[
        pltpu.VMEM((EPWP,), jnp.int32),
        pltpu.VMEM((EPWP,), jnp.int32),
        pltpu.VMEM((EPW,), jnp.int32),
        pltpu.VMEM((EPWP,), jnp.float32),
        pltpu.VMEM((NP,), jnp.float32),
        pltpu.VMEM((NP,), jnp.float32),
        pltpu.VMEM((G,), jnp.float32),
        pltpu.VMEM((NP,), jnp.float32),
        pltpu.SemaphoreType.DMA,
    ],
)
def _edge_agg(src, dst, ei, pe, p2, p3, pg, xw, out,
              src_v, dst_v, ei_v, pe_v, p2_v, p3_v, pg_v, acc_v, sem):
    _edge_body(src, dst, ei, pe, p2, p3, pg, xw, out,
               src_v, dst_v, ei_v, pe_v, p2_v, p3_v, pg_v, acc_v, sem)


def _reduce_body(p_ref, bn_ref, out_ref):
    out_ref[...] = (
        jnp.sum(p_ref[:, 0:N], axis=0, keepdims=True) + bn_ref[0, 0]
    )


def _reduce_partials(partials, bn):
    return pl.pallas_call(
        _reduce_body,
        in_specs=[
            pl.BlockSpec((NW, NP), lambda: (0, 0)),
            pl.BlockSpec(memory_space=pltpu.SMEM),
        ],
        out_specs=pl.BlockSpec((1, N), lambda: (0, 0)),
        out_shape=jax.ShapeDtypeStruct((1, N), jnp.float32),
    )(partials, bn)


def kernel(x, e, g, edges, edge_idx, node_idx, W_edge, b_edge, W_node, b_node):
    del node_idx
    be = b_edge.reshape(1, 1)
    bn = b_node.reshape(1, 1)

    pe, src2, dst2, p2o, p3o, xwo, pgp = _projections(
        e.T, edges, x, g, W_edge, W_node, be)

    partials = _edge_agg(src2, dst2, edge_idx, pe, p2o, p3o, pgp, xwo)
    out = _reduce_partials(partials, bn)
    return out.reshape(N, 1)


# speedup vs baseline: 57.7839x; 1.0414x over previous
import functools

import jax
import jax.numpy as jnp
from jax import lax
from jax.experimental import pallas as pl
from jax.experimental.pallas import tpu as pltpu
from jax.experimental.pallas import tpu_sc as plsc

N = 10000
E = 320000
D = 128
DE = 16
G = 16

NC = 2
NS = 16
NW = NC * NS
EPW = E // NW
LANES = 16
NVECS = N // LANES
VECS = EPW // LANES

ROWS = 10
EB = E // ROWS
EPWP = EPW + 112
NP = N + 112


def _proj_body(et_ref, edg_ref, x_ref, g_ref, wet_ref, wnt_ref,
               wns_ref, be_ref,
               pe_ref, src_ref, dst_ref, p2_ref, p3_ref, xw_ref, pg_ref):
    i = pl.program_id(0)
    wl = wns_ref[0, D]
    wes = wet_ref[...] * wl

    y = lax.dot_general(wes[:, 0:DE], et_ref[...],
                        (((1,), (0,)), ((), ())),
                        preferred_element_type=jnp.float32)
    pe_ref[...] = y + be_ref[0, 0] * wl

    src_ref[...] = edg_ref[0:1, :]
    dst_ref[...] = edg_ref[1:2, :]

    @pl.when(i == 0)
    def _():
        wxt = jnp.concatenate(
            [wes[:, DE:DE + D], wes[:, DE + D:DE + 2 * D],
             wnt_ref[:, 0:D]], axis=0)
        xp = lax.dot_general(wxt, x_ref[...],
                             (((1,), (1,)), ((), ())),
                             preferred_element_type=jnp.float32)
        p2_ref[0:1, 0:N] = xp[0:1, :]
        p3_ref[0:1, 0:N] = xp[1:2, :]
        xw_ref[0:1, 0:N] = xp[2:3, :]
        pg_ref[...] = lax.dot_general(wes[:, DE + 2 * D:DE + 2 * D + 64],
                                      g_ref[...],
                                      (((1,), (1,)), ((), ())),
                                      preferred_element_type=jnp.float32)


def _projections(et, edges, x, g, W_edge, W_node, be):
    return pl.pallas_call(
        _proj_body,
        grid=(ROWS,),
        in_specs=[
            pl.BlockSpec((DE, EB), lambda i: (0, i)),
            pl.BlockSpec((2, EB), lambda i: (0, i)),
            pl.BlockSpec((N, D), lambda i: (0, 0)),
            pl.BlockSpec((G, 64), lambda i: (0, 0)),
            pl.BlockSpec((1, DE + 2 * D + 64), lambda i: (0, 0)),
            pl.BlockSpec((1, D + 1), lambda i: (0, 0)),
            pl.BlockSpec(memory_space=pltpu.SMEM),
            pl.BlockSpec(memory_space=pltpu.SMEM),
        ],
        out_specs=[
            pl.BlockSpec((1, EB), lambda i: (0, i)),
            pl.BlockSpec((1, EB), lambda i: (0, i)),
            pl.BlockSpec((1, EB), lambda i: (0, i)),
            pl.BlockSpec((1, NP), lambda i: (0, 0)),
            pl.BlockSpec((1, NP), lambda i: (0, 0)),
            pl.BlockSpec((1, NP), lambda i: (0, 0)),
            pl.BlockSpec((1, G), lambda i: (0, 0)),
        ],
        out_shape=[
            jax.ShapeDtypeStruct((1, E), jnp.float32),
            jax.ShapeDtypeStruct((1, E), jnp.int32),
            jax.ShapeDtypeStruct((1, E), jnp.int32),
            jax.ShapeDtypeStruct((1, NP), jnp.float32),
            jax.ShapeDtypeStruct((1, NP), jnp.float32),
            jax.ShapeDtypeStruct((1, NP), jnp.float32),
            jax.ShapeDtypeStruct((1, G), jnp.float32),
        ],
    )(et, edges, x, g, W_edge.T, W_node.T, W_node.T, be)


def _edge_body(src_hbm, dst_hbm, ei_hbm, pe_hbm, p2_hbm, p3_hbm, pg_hbm,
               xw_hbm, out_hbm,
               src_v, dst_v, ei_v, pe_v, p2_v, p3_v, pg_v, acc_v, sem):
    wid = lax.axis_index("s") * NC + lax.axis_index("c")
    base = wid * EPW
    col0 = pl.multiple_of((base // 128) * 128, 128)
    delta = base - col0

    cps = [
        pltpu.async_copy(src_hbm.at[0, pl.ds(col0, EPWP)], src_v, sem),
        pltpu.async_copy(dst_hbm.at[0, pl.ds(col0, EPWP)], dst_v, sem),
        pltpu.async_copy(ei_hbm.at[pl.ds(base, EPW)], ei_v, sem),
        pltpu.async_copy(pe_hbm.at[0, pl.ds(col0, EPWP)], pe_v, sem),
        pltpu.async_copy(p2_hbm.at[0, pl.ds(0, NP)], p2_v, sem),
        pltpu.async_copy(p3_hbm.at[0, pl.ds(0, NP)], p3_v, sem),
        pltpu.async_copy(pg_hbm.at[0, pl.ds(0, G)], pg_v, sem),
    ]

    @pl.when(wid == 0)
    def _():
        pltpu.sync_copy(xw_hbm.at[0, pl.ds(0, NP)], acc_v)

    @pl.when(wid != 0)
    def _():
        zeros = jnp.zeros((LANES,), jnp.float32)

        @plsc.parallel_loop(0, NP // LANES, 1, unroll=8)
        def zbody(j):
            acc_v[pl.ds(j * LANES, LANES)] = zeros

    for c in cps:
        c.wait()

    @plsc.parallel_loop(0, VECS, 1, unroll=8)
    def body(j):
        sl = pl.ds(delta + j * LANES, LANES)
        s = src_v[sl]
        d = dst_v[sl]
        gi = ei_v[pl.ds(j * LANES, LANES)]
        v = pe_v[sl]
        v = v + plsc.load_gather(p2_v, [s])
        v = v + plsc.load_gather(p3_v, [d])
        v = v + plsc.load_gather(pg_v, [gi])
        plsc.addupdate_scatter(acc_v, [d], v)

    pltpu.sync_copy(acc_v, out_hbm.at[wid])


@functools.partial(
    pl.kernel,
    out_type=jax.ShapeDtypeStruct((NW, NP), jnp.float32),
    mesh=plsc.VectorSubcoreMesh(core_axis_name="c", subcore_axis_name="s",
                                num_cores=NC, num_subcores=NS),
    compiler_params=pltpu.CompilerParams(needs_layout_passes=False),
    scratch_types=[
        pltpu.VMEM((EPWP,), jnp.int32),
        pltpu.VMEM((EPWP,), jnp.int32),
        pltpu.VMEM((EPW,), jnp.int32),
        pltpu.VMEM((EPWP,), jnp.float32),
        pltpu.VMEM((NP,), jnp.float32),
        pltpu.VMEM((NP,), jnp.float32),
        pltpu.VMEM((G,), jnp.float32),
        pltpu.VMEM((NP,), jnp.float32),
        pltpu.SemaphoreType.DMA,
    ],
)
def _edge_agg(src, dst, ei, pe, p2, p3, pg, xw, out,
              src_v, dst_v, ei_v, pe_v, p2_v, p3_v, pg_v, acc_v, sem):
    _edge_body(src, dst, ei, pe, p2, p3, pg, xw, out,
               src_v, dst_v, ei_v, pe_v, p2_v, p3_v, pg_v, acc_v, sem)


def _reduce_body(p_ref, bn_ref, out_ref):
    out_ref[...] = (
        jnp.sum(p_ref[:, 0:N], axis=0, keepdims=True) + bn_ref[0, 0]
    )


def _reduce_partials(partials, bn):
    return pl.pallas_call(
        _reduce_body,
        in_specs=[
            pl.BlockSpec((NW, NP), lambda: (0, 0)),
            pl.BlockSpec(memory_space=pltpu.SMEM),
        ],
        out_specs=pl.BlockSpec((1, N), lambda: (0, 0)),
        out_shape=jax.ShapeDtypeStruct((1, N), jnp.float32),
    )(partials, bn)


def kernel(x, e, g, edges, edge_idx, node_idx, W_edge, b_edge, W_node, b_node):
    del node_idx
    be = b_edge.reshape(1, 1)
    bn = b_node.reshape(1, 1)

    pe, src2, dst2, p2o, p3o, xwo, pgp = _projections(
        e.T, edges, x, g, W_edge, W_node, be)

    partials = _edge_agg(src2, dst2, edge_idx, pe, p2o, p3o, pgp, xwo)
    out = _reduce_partials(partials, bn)
    return out.reshape(N, 1)


# speedup vs baseline: 60.6331x; 1.0493x over previous
import functools

import jax
import jax.numpy as jnp
from jax import lax
from jax.experimental import pallas as pl
from jax.experimental.pallas import tpu as pltpu
from jax.experimental.pallas import tpu_sc as plsc

N = 10000
E = 320000
D = 128
DE = 16
G = 16

NC = 2
NS = 16
NW = NC * NS
EPW = E // NW
LANES = 16
NVECS = N // LANES
VECS = EPW // LANES

ROWS = 5
EB = E // ROWS
EPWP = EPW + 112
NP = N + 112


def _proj_body(et_ref, edg_ref, x_ref, g_ref, wet_ref, wnt_ref,
               wns_ref, be_ref,
               pe_ref, src_ref, dst_ref, p2_ref, p3_ref, xw_ref, pg_ref):
    i = pl.program_id(0)
    wl = wns_ref[0, D]
    wes = wet_ref[...] * wl

    y = lax.dot_general(wes[:, 0:DE], et_ref[...],
                        (((1,), (0,)), ((), ())),
                        preferred_element_type=jnp.float32)
    pe_ref[...] = y + be_ref[0, 0] * wl

    src_ref[...] = edg_ref[0:1, :]
    dst_ref[...] = edg_ref[1:2, :]

    @pl.when(i == 0)
    def _():
        wxt = jnp.concatenate(
            [wes[:, DE:DE + D], wes[:, DE + D:DE + 2 * D],
             wnt_ref[:, 0:D]], axis=0)
        xp = lax.dot_general(wxt, x_ref[...],
                             (((1,), (1,)), ((), ())),
                             preferred_element_type=jnp.float32)
        p2_ref[0:1, 0:N] = xp[0:1, :]
        p3_ref[0:1, 0:N] = xp[1:2, :]
        xw_ref[0:1, 0:N] = xp[2:3, :]
        pg_ref[...] = lax.dot_general(wes[:, DE + 2 * D:DE + 2 * D + 64],
                                      g_ref[...],
                                      (((1,), (1,)), ((), ())),
                                      preferred_element_type=jnp.float32)


def _projections(et, edges, x, g, W_edge, W_node, be):
    return pl.pallas_call(
        _proj_body,
        grid=(ROWS,),
        in_specs=[
            pl.BlockSpec((DE, EB), lambda i: (0, i)),
            pl.BlockSpec((2, EB), lambda i: (0, i)),
            pl.BlockSpec((N, D), lambda i: (0, 0)),
            pl.BlockSpec((G, 64), lambda i: (0, 0)),
            pl.BlockSpec((1, DE + 2 * D + 64), lambda i: (0, 0)),
            pl.BlockSpec((1, D + 1), lambda i: (0, 0)),
            pl.BlockSpec(memory_space=pltpu.SMEM),
            pl.BlockSpec(memory_space=pltpu.SMEM),
        ],
        out_specs=[
            pl.BlockSpec((1, EB), lambda i: (0, i)),
            pl.BlockSpec((1, EB), lambda i: (0, i)),
            pl.BlockSpec((1, EB), lambda i: (0, i)),
            pl.BlockSpec((1, NP), lambda i: (0, 0)),
            pl.BlockSpec((1, NP), lambda i: (0, 0)),
            pl.BlockSpec((1, NP), lambda i: (0, 0)),
            pl.BlockSpec((1, G), lambda i: (0, 0)),
        ],
        out_shape=[
            jax.ShapeDtypeStruct((1, E), jnp.float32),
            jax.ShapeDtypeStruct((1, E), jnp.int32),
            jax.ShapeDtypeStruct((1, E), jnp.int32),
            jax.ShapeDtypeStruct((1, NP), jnp.float32),
            jax.ShapeDtypeStruct((1, NP), jnp.float32),
            jax.ShapeDtypeStruct((1, NP), jnp.float32),
            jax.ShapeDtypeStruct((1, G), jnp.float32),
        ],
    )(et, edges, x, g, W_edge.T, W_node.T, W_node.T, be)


def _edge_body(src_hbm, dst_hbm, ei_hbm, pe_hbm, p2_hbm, p3_hbm, pg_hbm,
               xw_hbm, out_hbm,
               src_v, dst_v, ei_v, pe_v, p2_v, p3_v, pg_v, acc_v, sem):
    wid = lax.axis_index("s") * NC + lax.axis_index("c")
    base = wid * EPW
    col0 = pl.multiple_of((base // 128) * 128, 128)
    delta = base - col0

    cps = [
        pltpu.async_copy(src_hbm.at[0, pl.ds(col0, EPWP)], src_v, sem),
        pltpu.async_copy(dst_hbm.at[0, pl.ds(col0, EPWP)], dst_v, sem),
        pltpu.async_copy(ei_hbm.at[pl.ds(base, EPW)], ei_v, sem),
        pltpu.async_copy(pe_hbm.at[0, pl.ds(col0, EPWP)], pe_v, sem),
        pltpu.async_copy(p2_hbm.at[0, pl.ds(0, NP)], p2_v, sem),
        pltpu.async_copy(p3_hbm.at[0, pl.ds(0, NP)], p3_v, sem),
        pltpu.async_copy(pg_hbm.at[0, pl.ds(0, G)], pg_v, sem),
    ]

    @pl.when(wid == 0)
    def _():
        pltpu.sync_copy(xw_hbm.at[0, pl.ds(0, NP)], acc_v)

    @pl.when(wid != 0)
    def _():
        zeros = jnp.zeros((LANES,), jnp.float32)

        @plsc.parallel_loop(0, NP // LANES, 1, unroll=8)
        def zbody(j):
            acc_v[pl.ds(j * LANES, LANES)] = zeros

    for c in cps:
        c.wait()

    @plsc.parallel_loop(0, VECS, 1, unroll=8)
    def body(j):
        sl = pl.ds(delta + j * LANES, LANES)
        s = src_v[sl]
        d = dst_v[sl]
        gi = ei_v[pl.ds(j * LANES, LANES)]
        v = pe_v[sl]
        v = v + plsc.load_gather(p2_v, [s])
        v = v + plsc.load_gather(p3_v, [d])
        v = v + plsc.load_gather(pg_v, [gi])
        plsc.addupdate_scatter(acc_v, [d], v)

    pltpu.sync_copy(acc_v, out_hbm.at[wid])


@functools.partial(
    pl.kernel,
    out_type=jax.ShapeDtypeStruct((NW, NP), jnp.float32),
    mesh=plsc.VectorSubcoreMesh(core_axis_name="c", subcore_axis_name="s",
                                num_cores=NC, num_subcores=NS),
    compiler_params=pltpu.CompilerParams(needs_layout_passes=False),
    scratch_types=[
        pltpu.VMEM((EPWP,), jnp.int32),
        pltpu.VMEM((EPWP,), jnp.int32),
        pltpu.VMEM((EPW,), jnp.int32),
        pltpu.VMEM((EPWP,), jnp.float32),
        pltpu.VMEM((NP,), jnp.float32),
        pltpu.VMEM((NP,), jnp.float32),
        pltpu.VMEM((G,), jnp.float32),
        pltpu.VMEM((NP,), jnp.float32),
        pltpu.SemaphoreType.DMA,
    ],
)
def _edge_agg(src, dst, ei, pe, p2, p3, pg, xw, out,
              src_v, dst_v, ei_v, pe_v, p2_v, p3_v, pg_v, acc_v, sem):
    _edge_body(src, dst, ei, pe, p2, p3, pg, xw, out,
               src_v, dst_v, ei_v, pe_v, p2_v, p3_v, pg_v, acc_v, sem)


def _reduce_body(p_ref, bn_ref, out_ref):
    out_ref[...] = (
        jnp.sum(p_ref[:, 0:N], axis=0, keepdims=True) + bn_ref[0, 0]
    )


def _reduce_partials(partials, bn):
    return pl.pallas_call(
        _reduce_body,
        in_specs=[
            pl.BlockSpec((NW, NP), lambda: (0, 0)),
            pl.BlockSpec(memory_space=pltpu.SMEM),
        ],
        out_specs=pl.BlockSpec((1, N), lambda: (0, 0)),
        out_shape=jax.ShapeDtypeStruct((1, N), jnp.float32),
    )(partials, bn)


def kernel(x, e, g, edges, edge_idx, node_idx, W_edge, b_edge, W_node, b_node):
    del node_idx
    be = b_edge.reshape(1, 1)
    bn = b_node.reshape(1, 1)

    pe, src2, dst2, p2o, p3o, xwo, pgp = _projections(
        e.T, edges, x, g, W_edge, W_node, be)

    partials = _edge_agg(src2, dst2, edge_idx, pe, p2o, p3o, pgp, xwo)
    out = _reduce_partials(partials, bn)
    return out.reshape(N, 1)


# speedup vs baseline: 60.7069x; 1.0012x over previous
import functools

import jax
import jax.numpy as jnp
from jax import lax
from jax.experimental import pallas as pl
from jax.experimental.pallas import tpu as pltpu
from jax.experimental.pallas import tpu_sc as plsc

N = 10000
E = 320000
D = 128
DE = 16
G = 16

NC = 2
NS = 16
NW = NC * NS
EPW = E // NW
LANES = 16
NVECS = N // LANES
VECS = EPW // LANES

ROWS = 5
EB = E // ROWS
EPWP = EPW + 112
NP = N + 112
CH0 = 4992
CH1 = EPW - CH0
CW = 5120


def _proj_body(et_ref, edg_ref, x_ref, g_ref, wet_ref, wnt_ref,
               wns_ref, be_ref,
               pe_ref, src_ref, dst_ref, p2_ref, p3_ref, xw_ref, pg_ref):
    i = pl.program_id(0)
    wl = wns_ref[0, D]
    wes = wet_ref[...] * wl

    y = lax.dot_general(wes[:, 0:DE], et_ref[...],
                        (((1,), (0,)), ((), ())),
                        preferred_element_type=jnp.float32)
    pe_ref[...] = y + be_ref[0, 0] * wl

    src_ref[...] = edg_ref[0:1, :]
    dst_ref[...] = edg_ref[1:2, :]

    @pl.when(i == 0)
    def _():
        wxt = jnp.concatenate(
            [wes[:, DE:DE + D], wes[:, DE + D:DE + 2 * D],
             wnt_ref[:, 0:D]], axis=0)
        xp = lax.dot_general(wxt, x_ref[...],
                             (((1,), (1,)), ((), ())),
                             preferred_element_type=jnp.float32)
        p2_ref[0:1, 0:N] = xp[0:1, :]
        p3_ref[0:1, 0:N] = xp[1:2, :]
        xw_ref[0:1, 0:N] = xp[2:3, :]
        pg_ref[...] = lax.dot_general(wes[:, DE + 2 * D:DE + 2 * D + 64],
                                      g_ref[...],
                                      (((1,), (1,)), ((), ())),
                                      preferred_element_type=jnp.float32)


def _projections(et, edges, x, g, W_edge, W_node, be):
    return pl.pallas_call(
        _proj_body,
        grid=(ROWS,),
        in_specs=[
            pl.BlockSpec((DE, EB), lambda i: (0, i)),
            pl.BlockSpec((2, EB), lambda i: (0, i)),
            pl.BlockSpec((N, D), lambda i: (0, 0)),
            pl.BlockSpec((G, 64), lambda i: (0, 0)),
            pl.BlockSpec((1, DE + 2 * D + 64), lambda i: (0, 0)),
            pl.BlockSpec((1, D + 1), lambda i: (0, 0)),
            pl.BlockSpec(memory_space=pltpu.SMEM),
            pl.BlockSpec(memory_space=pltpu.SMEM),
        ],
        out_specs=[
            pl.BlockSpec((1, EB), lambda i: (0, i)),
            pl.BlockSpec((1, EB), lambda i: (0, i)),
            pl.BlockSpec((1, EB), lambda i: (0, i)),
            pl.BlockSpec((1, NP), lambda i: (0, 0)),
            pl.BlockSpec((1, NP), lambda i: (0, 0)),
            pl.BlockSpec((1, NP), lambda i: (0, 0)),
            pl.BlockSpec((1, G), lambda i: (0, 0)),
        ],
        out_shape=[
            jax.ShapeDtypeStruct((1, E), jnp.float32),
            jax.ShapeDtypeStruct((1, E), jnp.int32),
            jax.ShapeDtypeStruct((1, E), jnp.int32),
            jax.ShapeDtypeStruct((1, NP), jnp.float32),
            jax.ShapeDtypeStruct((1, NP), jnp.float32),
            jax.ShapeDtypeStruct((1, NP), jnp.float32),
            jax.ShapeDtypeStruct((1, G), jnp.float32),
        ],
    )(et, edges, x, g, W_edge.T, W_node.T, W_node.T, be)


def _edge_body(src_hbm, dst_hbm, ei_hbm, pe_hbm, p2_hbm, p3_hbm, pg_hbm,
               xw_hbm, out_hbm,
               src_v, dst_v, ei_v, pe_v, p2_v, p3_v, pg_v, acc_v,
               sem, sem2):
    wid = lax.axis_index("s") * NC + lax.axis_index("c")
    base = wid * EPW

    bases = (base, base + CH0)
    sizes = (CH0, CH1)
    cols = []
    deltas = []
    for c in range(2):
        c0 = pl.multiple_of((bases[c] // 128) * 128, 128)
        cols.append(c0)
        deltas.append(bases[c] - c0)

    def fire(c, s):
        off = c * CH0
        return [
            pltpu.async_copy(src_hbm.at[0, pl.ds(cols[c], CW)],
                             src_v.at[pl.ds(c * CW, CW)], s),
            pltpu.async_copy(dst_hbm.at[0, pl.ds(cols[c], CW)],
                             dst_v.at[pl.ds(c * CW, CW)], s),
            pltpu.async_copy(ei_hbm.at[pl.ds(bases[c], sizes[c])],
                             ei_v.at[pl.ds(off, sizes[c])], s),
            pltpu.async_copy(pe_hbm.at[0, pl.ds(cols[c], CW)],
                             pe_v.at[pl.ds(c * CW, CW)], s),
        ]

    cps = fire(0, sem) + [
        pltpu.async_copy(p2_hbm.at[0, pl.ds(0, NP)], p2_v, sem),
        pltpu.async_copy(p3_hbm.at[0, pl.ds(0, NP)], p3_v, sem),
        pltpu.async_copy(pg_hbm.at[0, pl.ds(0, G)], pg_v, sem),
    ]
    cps2 = fire(1, sem2)

    @pl.when(wid == 0)
    def _():
        pltpu.sync_copy(xw_hbm.at[0, pl.ds(0, NP)], acc_v)

    @pl.when(wid != 0)
    def _():
        zeros = jnp.zeros((LANES,), jnp.float32)

        @plsc.parallel_loop(0, NP // LANES, 1, unroll=8)
        def zbody(j):
            acc_v[pl.ds(j * LANES, LANES)] = zeros

    for c in cps:
        c.wait()

    def run_chunk(c, nvec):
        delta = deltas[c]
        cbase = c * CW
        ebase = c * CH0

        @plsc.parallel_loop(0, nvec, 1, unroll=8)
        def body(j):
            sl = pl.ds(cbase + delta + j * LANES, LANES)
            s = src_v[sl]
            d = dst_v[sl]
            gi = ei_v[pl.ds(ebase + j * LANES, LANES)]
            v = pe_v[sl]
            v = v + plsc.load_gather(p2_v, [s])
            v = v + plsc.load_gather(p3_v, [d])
            v = v + plsc.load_gather(pg_v, [gi])
            plsc.addupdate_scatter(acc_v, [d], v)

    run_chunk(0, CH0 // LANES)
    for c in cps2:
        c.wait()
    run_chunk(1, CH1 // LANES)

    pltpu.sync_copy(acc_v, out_hbm.at[wid])


@functools.partial(
    pl.kernel,
    out_type=jax.ShapeDtypeStruct((NW, NP), jnp.float32),
    mesh=plsc.VectorSubcoreMesh(core_axis_name="c", subcore_axis_name="s",
                                num_cores=NC, num_subcores=NS),
    compiler_params=pltpu.CompilerParams(needs_layout_passes=False),
    scratch_types=[
        pltpu.VMEM((2 * CW,), jnp.int32),
        pltpu.VMEM((2 * CW,), jnp.int32),
        pltpu.VMEM((EPW,), jnp.int32),
        pltpu.VMEM((2 * CW,), jnp.float32),
        pltpu.VMEM((NP,), jnp.float32),
        pltpu.VMEM((NP,), jnp.float32),
        pltpu.VMEM((G,), jnp.float32),
        pltpu.VMEM((NP,), jnp.float32),
        pltpu.SemaphoreType.DMA,
        pltpu.SemaphoreType.DMA,
    ],
)
def _edge_agg(src, dst, ei, pe, p2, p3, pg, xw, out,
              src_v, dst_v, ei_v, pe_v, p2_v, p3_v, pg_v, acc_v, sem, sem2):
    _edge_body(src, dst, ei, pe, p2, p3, pg, xw, out,
               src_v, dst_v, ei_v, pe_v, p2_v, p3_v, pg_v, acc_v, sem, sem2)


def _reduce_body(p_ref, bn_ref, out_ref):
    out_ref[...] = (
        jnp.sum(p_ref[:, 0:N], axis=0, keepdims=True) + bn_ref[0, 0]
    )


def _reduce_partials(partials, bn):
    return pl.pallas_call(
        _reduce_body,
        in_specs=[
            pl.BlockSpec((NW, NP), lambda: (0, 0)),
            pl.BlockSpec(memory_space=pltpu.SMEM),
        ],
        out_specs=pl.BlockSpec((1, N), lambda: (0, 0)),
        out_shape=jax.ShapeDtypeStruct((1, N), jnp.float32),
    )(partials, bn)


def kernel(x, e, g, edges, edge_idx, node_idx, W_edge, b_edge, W_node, b_node):
    del node_idx
    be = b_edge.reshape(1, 1)
    bn = b_node.reshape(1, 1)

    pe, src2, dst2, p2o, p3o, xwo, pgp = _projections(
        e.T, edges, x, g, W_edge, W_node, be)

    partials = _edge_agg(src2, dst2, edge_idx, pe, p2o, p3o, pgp, xwo)
    out = _reduce_partials(partials, bn)
    return out.reshape(N, 1)


# speedup vs baseline: 61.4119x; 1.0116x over previous
import functools

import jax
import jax.numpy as jnp
from jax import lax
from jax.experimental import pallas as pl
from jax.experimental.pallas import tpu as pltpu
from jax.experimental.pallas import tpu_sc as plsc

N = 10000
E = 320000
D = 128
DE = 16
G = 16

NC = 2
NS = 16
NW = NC * NS
EPW = E // NW
LANES = 16
NVECS = N // LANES
VECS = EPW // LANES

ROWS = 4
EB = E // ROWS
EPWP = EPW + 112
NP = N + 112
CH0 = 4992
CH1 = EPW - CH0
CW = 5120


def _proj_body(et_ref, edg_ref, x_ref, g_ref, wet_ref, wnt_ref,
               wns_ref, be_ref,
               pe_ref, src_ref, dst_ref, p2_ref, p3_ref, xw_ref, pg_ref):
    i = pl.program_id(0)
    wl = wns_ref[0, D]
    wes = wet_ref[...] * wl

    y = lax.dot_general(wes[:, 0:DE], et_ref[...],
                        (((1,), (0,)), ((), ())),
                        preferred_element_type=jnp.float32)
    pe_ref[...] = y + be_ref[0, 0] * wl

    src_ref[...] = edg_ref[0:1, :]
    dst_ref[...] = edg_ref[1:2, :]

    @pl.when(i == 0)
    def _():
        wxt = jnp.concatenate(
            [wes[:, DE:DE + D], wes[:, DE + D:DE + 2 * D],
             wnt_ref[:, 0:D]], axis=0)
        xp = lax.dot_general(wxt, x_ref[...],
                             (((1,), (1,)), ((), ())),
                             preferred_element_type=jnp.float32)
        p2_ref[0:1, 0:N] = xp[0:1, :]
        p3_ref[0:1, 0:N] = xp[1:2, :]
        xw_ref[0:1, 0:N] = xp[2:3, :]
        pg_ref[...] = lax.dot_general(wes[:, DE + 2 * D:DE + 2 * D + 64],
                                      g_ref[...],
                                      (((1,), (1,)), ((), ())),
                                      preferred_element_type=jnp.float32)


def _projections(et, edges, x, g, W_edge, W_node, be):
    return pl.pallas_call(
        _proj_body,
        grid=(ROWS,),
        in_specs=[
            pl.BlockSpec((DE, EB), lambda i: (0, i)),
            pl.BlockSpec((2, EB), lambda i: (0, i)),
            pl.BlockSpec((N, D), lambda i: (0, 0)),
            pl.BlockSpec((G, 64), lambda i: (0, 0)),
            pl.BlockSpec((1, DE + 2 * D + 64), lambda i: (0, 0)),
            pl.BlockSpec((1, D + 1), lambda i: (0, 0)),
            pl.BlockSpec(memory_space=pltpu.SMEM),
            pl.BlockSpec(memory_space=pltpu.SMEM),
        ],
        out_specs=[
            pl.BlockSpec((1, EB), lambda i: (0, i)),
            pl.BlockSpec((1, EB), lambda i: (0, i)),
            pl.BlockSpec((1, EB), lambda i: (0, i)),
            pl.BlockSpec((1, NP), lambda i: (0, 0)),
            pl.BlockSpec((1, NP), lambda i: (0, 0)),
            pl.BlockSpec((1, NP), lambda i: (0, 0)),
            pl.BlockSpec((1, G), lambda i: (0, 0)),
        ],
        out_shape=[
            jax.ShapeDtypeStruct((1, E), jnp.float32),
            jax.ShapeDtypeStruct((1, E), jnp.int32),
            jax.ShapeDtypeStruct((1, E), jnp.int32),
            jax.ShapeDtypeStruct((1, NP), jnp.float32),
            jax.ShapeDtypeStruct((1, NP), jnp.float32),
            jax.ShapeDtypeStruct((1, NP), jnp.float32),
            jax.ShapeDtypeStruct((1, G), jnp.float32),
        ],
    )(et, edges, x, g, W_edge.T, W_node.T, W_node.T, be)


def _edge_body(src_hbm, dst_hbm, ei_hbm, pe_hbm, p2_hbm, p3_hbm, pg_hbm,
               xw_hbm, out_hbm,
               src_v, dst_v, ei_v, pe_v, p2_v, p3_v, pg_v, acc_v,
               sem, sem2):
    wid = lax.axis_index("s") * NC + lax.axis_index("c")
    base = wid * EPW

    bases = (base, base + CH0)
    sizes = (CH0, CH1)
    cols = []
    deltas = []
    for c in range(2):
        c0 = pl.multiple_of((bases[c] // 128) * 128, 128)
        cols.append(c0)
        deltas.append(bases[c] - c0)

    def fire(c, s):
        off = c * CH0
        return [
            pltpu.async_copy(src_hbm.at[0, pl.ds(cols[c], CW)],
                             src_v.at[pl.ds(c * CW, CW)], s),
            pltpu.async_copy(dst_hbm.at[0, pl.ds(cols[c], CW)],
                             dst_v.at[pl.ds(c * CW, CW)], s),
            pltpu.async_copy(ei_hbm.at[pl.ds(bases[c], sizes[c])],
                             ei_v.at[pl.ds(off, sizes[c])], s),
            pltpu.async_copy(pe_hbm.at[0, pl.ds(cols[c], CW)],
                             pe_v.at[pl.ds(c * CW, CW)], s),
        ]

    cps = fire(0, sem) + [
        pltpu.async_copy(p2_hbm.at[0, pl.ds(0, NP)], p2_v, sem),
        pltpu.async_copy(p3_hbm.at[0, pl.ds(0, NP)], p3_v, sem),
        pltpu.async_copy(pg_hbm.at[0, pl.ds(0, G)], pg_v, sem),
    ]
    cps2 = fire(1, sem2)

    @pl.when(wid == 0)
    def _():
        pltpu.sync_copy(xw_hbm.at[0, pl.ds(0, NP)], acc_v)

    @pl.when(wid != 0)
    def _():
        zeros = jnp.zeros((LANES,), jnp.float32)

        @plsc.parallel_loop(0, NP // LANES, 1, unroll=8)
        def zbody(j):
            acc_v[pl.ds(j * LANES, LANES)] = zeros

    for c in cps:
        c.wait()

    def run_chunk(c, nvec):
        delta = deltas[c]
        cbase = c * CW
        ebase = c * CH0

        @plsc.parallel_loop(0, nvec, 1, unroll=8)
        def body(j):
            sl = pl.ds(cbase + delta + j * LANES, LANES)
            s = src_v[sl]
            d = dst_v[sl]
            gi = ei_v[pl.ds(ebase + j * LANES, LANES)]
            v = pe_v[sl]
            v = v + plsc.load_gather(p2_v, [s])
            v = v + plsc.load_gather(p3_v, [d])
            v = v + plsc.load_gather(pg_v, [gi])
            plsc.addupdate_scatter(acc_v, [d], v)

    run_chunk(0, CH0 // LANES)
    for c in cps2:
        c.wait()
    run_chunk(1, CH1 // LANES)

    pltpu.sync_copy(acc_v, out_hbm.at[wid])


@functools.partial(
    pl.kernel,
    out_type=jax.ShapeDtypeStruct((NW, NP), jnp.float32),
    mesh=plsc.VectorSubcoreMesh(core_axis_name="c", subcore_axis_name="s",
                                num_cores=NC, num_subcores=NS),
    compiler_params=pltpu.CompilerParams(needs_layout_passes=False),
    scratch_types=[
        pltpu.VMEM((2 * CW,), jnp.int32),
        pltpu.VMEM((2 * CW,), jnp.int32),
        pltpu.VMEM((EPW,), jnp.int32),
        pltpu.VMEM((2 * CW,), jnp.float32),
        pltpu.VMEM((NP,), jnp.float32),
        pltpu.VMEM((NP,), jnp.float32),
        pltpu.VMEM((G,), jnp.float32),
        pltpu.VMEM((NP,), jnp.float32),
        pltpu.SemaphoreType.DMA,
        pltpu.SemaphoreType.DMA,
    ],
)
def _edge_agg(src, dst, ei, pe, p2, p3, pg, xw, out,
              src_v, dst_v, ei_v, pe_v, p2_v, p3_v, pg_v, acc_v, sem, sem2):
    _edge_body(src, dst, ei, pe, p2, p3, pg, xw, out,
               src_v, dst_v, ei_v, pe_v, p2_v, p3_v, pg_v, acc_v, sem, sem2)


def _reduce_body(p_ref, bn_ref, out_ref):
    out_ref[...] = (
        jnp.sum(p_ref[:, 0:N], axis=0, keepdims=True) + bn_ref[0, 0]
    )


def _reduce_partials(partials, bn):
    return pl.pallas_call(
        _reduce_body,
        in_specs=[
            pl.BlockSpec((NW, NP), lambda: (0, 0)),
            pl.BlockSpec(memory_space=pltpu.SMEM),
        ],
        out_specs=pl.BlockSpec((1, N), lambda: (0, 0)),
        out_shape=jax.ShapeDtypeStruct((1, N), jnp.float32),
    )(partials, bn)


def kernel(x, e, g, edges, edge_idx, node_idx, W_edge, b_edge, W_node, b_node):
    del node_idx
    be = b_edge.reshape(1, 1)
    bn = b_node.reshape(1, 1)

    pe, src2, dst2, p2o, p3o, xwo, pgp = _projections(
        e.T, edges, x, g, W_edge, W_node, be)

    partials = _edge_agg(src2, dst2, edge_idx, pe, p2o, p3o, pgp, xwo)
    out = _reduce_partials(partials, bn)
    return out.reshape(N, 1)
